# baseline (speedup 1.0000x reference)
import functools

import jax
import jax.numpy as jnp
from jax import lax
from jax.experimental import pallas as pl
from jax.experimental.pallas import tpu as pltpu
from jax.experimental.pallas import tpu_sc as plsc

NCLUST = 10
DIN = 1024
DHID = 512
DATT = 256
NTOK = 50000
BLK = 256
NBLK = 208
NPAD = NBLK * BLK
NTOKP = 50176
CHUNK = NTOKP // 32


def _tc_kernel(blk_cid_ref, blk_valid_ref,
               x_ref, w1_ref, b1_ref, w2_ref, b2_ref, counts_ref,
               wfc_ref, bfc_ref, wa_ref, ba_ref, wb_ref, bb_ref,
               wc_ref, bc_ref, wrho_ref, brho_ref, wcls_ref, bcls_ref,
               logits_ref, prob_ref, yhat_ref, acc_ref):
    i = pl.program_id(0)
    cid = blk_cid_ref[i]
    nvalid = blk_valid_ref[i]

    @pl.when(i == 0)
    def _init():
        acc_ref[...] = jnp.zeros_like(acc_ref)

    x = x_ref[...]
    h = jnp.dot(x, w1_ref[cid], preferred_element_type=jnp.float32)
    h = jnp.maximum(h + b1_ref[pl.ds(cid, 1), :], 0.0)
    h = jnp.dot(h, w2_ref[cid], preferred_element_type=jnp.float32)
    h = jnp.maximum(h + b2_ref[pl.ds(cid, 1), :], 0.0)
    rows = lax.broadcasted_iota(jnp.int32, (BLK, 1), 0)
    h = jnp.where(rows < nvalid, h, 0.0)
    part = jnp.sum(h, axis=0, keepdims=True)
    acc_ref[pl.ds(cid, 1), :] += part

    @pl.when(i == NBLK - 1)
    def _head():
        lane10 = lax.broadcasted_iota(jnp.int32, (16, 1), 0) < NCLUST
        h_cluster = acc_ref[...] / counts_ref[...]
        h_path = jnp.dot(h_cluster, wfc_ref[...], preferred_element_type=jnp.float32)
        h_path = jnp.maximum(h_path + bfc_ref[...], 0.0)
        a = jnp.tanh(jnp.dot(h_path, wa_ref[...], preferred_element_type=jnp.float32) + ba_ref[...])
        b = jax.nn.sigmoid(jnp.dot(h_path, wb_ref[...], preferred_element_type=jnp.float32) + bb_ref[...])
        g = a * b
        scores = jnp.sum(g * wc_ref[...], axis=1, keepdims=True) + bc_ref[0, 0]
        scores = jnp.where(lane10, scores, -jnp.inf)
        m = jnp.max(scores, axis=0, keepdims=True)
        e = jnp.exp(scores - m)
        attn = e / jnp.sum(e, axis=0, keepdims=True)
        hp = jnp.sum(attn * h_path, axis=0, keepdims=True)
        hr = jnp.dot(hp, wrho_ref[...], preferred_element_type=jnp.float32)
        hr = jnp.maximum(hr + brho_ref[...], 0.0)
        logits = jnp.dot(hr, wcls_ref[...], preferred_element_type=jnp.float32) + bcls_ref[...]
        logits_ref[...] = logits
        lm = jnp.max(logits, axis=1, keepdims=True)
        le = jnp.exp(logits - lm)
        prob_ref[...] = le / jnp.sum(le, axis=1, keepdims=True)
        lane4 = lax.broadcasted_iota(jnp.int32, (1, 4), 1)
        yhat_ref[...] = jnp.min(
            jnp.where(logits >= lm, lane4, 9999), axis=1, keepdims=True)


def _run_tc(x_sorted, blk_cid, blk_valid, counts16, phi_W1, phi_b1, phi_W2,
            phi_b2, W_fc, b_fc, W_a, b_a, W_b, b_b, W_c, b_c, W_rho, b_rho,
            W_cls, b_cls):
    full = lambda *shape: pl.BlockSpec(shape, lambda i, *_: (0,) * len(shape))
    grid_spec = pltpu.PrefetchScalarGridSpec(
        num_scalar_prefetch=2,
        grid=(NBLK,),
        in_specs=[
            pl.BlockSpec((BLK, DIN), lambda i, *_: (i, 0)),
            full(NCLUST, DIN, DHID),
            full(NCLUST, DHID),
            full(NCLUST, DHID, DHID),
            full(NCLUST, DHID),
            full(16, 1),
            full(DHID, DHID), full(1, DHID),
            full(DHID, DATT), full(1, DATT),
            full(DHID, DATT), full(1, DATT),
            full(1, DATT), full(1, 1),
            full(DHID, DATT), full(1, DATT),
            full(DATT, 4), full(1, 4),
        ],
        out_specs=[full(1, 4), full(1, 4), full(1, 1)],
        scratch_shapes=[pltpu.VMEM((16, DHID), jnp.float32)],
    )
    return pl.pallas_call(
        _tc_kernel,
        grid_spec=grid_spec,
        out_shape=[
            jax.ShapeDtypeStruct((1, 4), jnp.float32),
            jax.ShapeDtypeStruct((1, 4), jnp.float32),
            jax.ShapeDtypeStruct((1, 1), jnp.int32),
        ],
    )(blk_cid, blk_valid, x_sorted, phi_W1, phi_b1, phi_W2, phi_b2, counts16,
      W_fc, b_fc.reshape(1, DHID), W_a, b_a.reshape(1, DATT),
      W_b, b_b.reshape(1, DATT), W_c.reshape(1, DATT), b_c.reshape(1, 1),
      W_rho, b_rho.reshape(1, DATT), W_cls, b_cls.reshape(1, 4))


def _route_host(cluster_id):
    cid = cluster_id.astype(jnp.int32)
    counts = jnp.zeros((NCLUST,), jnp.int32).at[cid].add(1)
    padded = ((counts + BLK - 1) // BLK) * BLK
    offsets = jnp.concatenate([jnp.zeros((1,), jnp.int32), jnp.cumsum(padded)])
    order = jnp.argsort(cid, stable=True)
    sorted_start = jnp.concatenate([jnp.zeros((1,), jnp.int32),
                                    jnp.cumsum(counts)])
    cid_sorted = cid[order]
    dest = (offsets[cid_sorted] + jnp.arange(NTOK, dtype=jnp.int32)
            - sorted_start[cid_sorted])
    perm = jnp.zeros((NPAD,), jnp.int32).at[dest].set(order.astype(jnp.int32))
    jb = jnp.arange(NBLK, dtype=jnp.int32) * BLK
    blk_cid = jnp.sum((jb[:, None] >= offsets[None, 1:]).astype(jnp.int32),
                      axis=1)
    blk_cid = jnp.minimum(blk_cid, NCLUST - 1)
    blk_valid = jnp.clip(counts[blk_cid] - (jb - offsets[blk_cid]), 0, BLK)
    counts16 = jnp.where(jnp.arange(16) < NCLUST,
                         jnp.concatenate([counts, jnp.ones((6,), jnp.int32)]),
                         1).astype(jnp.float32).reshape(16, 1)
    return perm, blk_cid, blk_valid, counts16


def kernel(data, cluster_id, phi_W1, phi_b1, phi_W2, phi_b2, W_fc, b_fc,
           W_a, b_a, W_b, b_b, W_c, b_c, W_rho, b_rho, W_cls, b_cls):
    perm, blk_cid, blk_valid, counts16 = _route_host(cluster_id)
    x_sorted = jnp.take(data, perm, axis=0)
    logits, prob, yhat = _run_tc(
        x_sorted, blk_cid, blk_valid, counts16, phi_W1, phi_b1, phi_W2,
        phi_b2, W_fc, b_fc, W_a, b_a, W_b, b_b, W_c, b_c, W_rho, b_rho,
        W_cls, b_cls)
    return (logits, prob, yhat)


# speedup vs baseline: 1.2030x; 1.2030x over previous
import functools

import jax
import jax.numpy as jnp
from jax import lax
from jax.experimental import pallas as pl
from jax.experimental.pallas import tpu as pltpu
from jax.experimental.pallas import tpu_sc as plsc

NCLUST = 10
DIN = 1024
DHID = 512
DATT = 256
NTOK = 50000
BLK = 256
NBLK = 208
NPAD = NBLK * BLK
NTOKP = 50176
CHUNK = NTOKP // 32


def _tc_kernel(blk_cid_ref, blk_valid_ref,
               x_ref, w1_ref, b1_ref, w2_ref, b2_ref, counts_ref,
               wfc_ref, bfc_ref, wa_ref, ba_ref, wb_ref, bb_ref,
               wc_ref, bc_ref, wrho_ref, brho_ref, wcls_ref, bcls_ref,
               logits_ref, prob_ref, yhat_ref, acc_ref):
    i = pl.program_id(0)
    cid = blk_cid_ref[i]
    nvalid = blk_valid_ref[i]

    @pl.when(i == 0)
    def _init():
        acc_ref[...] = jnp.zeros_like(acc_ref)

    x = x_ref[...]
    h = jnp.dot(x, w1_ref[cid], preferred_element_type=jnp.float32)
    h = jnp.maximum(h + b1_ref[pl.ds(cid, 1), :], 0.0)
    h = jnp.dot(h, w2_ref[cid], preferred_element_type=jnp.float32)
    h = jnp.maximum(h + b2_ref[pl.ds(cid, 1), :], 0.0)
    rows = lax.broadcasted_iota(jnp.int32, (BLK, 1), 0)
    h = jnp.where(rows < nvalid, h, 0.0)
    part = jnp.sum(h, axis=0, keepdims=True)
    acc_ref[pl.ds(cid, 1), :] += part

    @pl.when(i == NBLK - 1)
    def _head():
        lane10 = lax.broadcasted_iota(jnp.int32, (16, 1), 0) < NCLUST
        h_cluster = acc_ref[...] / counts_ref[...]
        h_path = jnp.dot(h_cluster, wfc_ref[...], preferred_element_type=jnp.float32)
        h_path = jnp.maximum(h_path + bfc_ref[...], 0.0)
        a = jnp.tanh(jnp.dot(h_path, wa_ref[...], preferred_element_type=jnp.float32) + ba_ref[...])
        b = jax.nn.sigmoid(jnp.dot(h_path, wb_ref[...], preferred_element_type=jnp.float32) + bb_ref[...])
        g = a * b
        scores = jnp.sum(g * wc_ref[...], axis=1, keepdims=True) + bc_ref[0, 0]
        scores = jnp.where(lane10, scores, -jnp.inf)
        m = jnp.max(scores, axis=0, keepdims=True)
        e = jnp.exp(scores - m)
        attn = e / jnp.sum(e, axis=0, keepdims=True)
        hp = jnp.sum(attn * h_path, axis=0, keepdims=True)
        hr = jnp.dot(hp, wrho_ref[...], preferred_element_type=jnp.float32)
        hr = jnp.maximum(hr + brho_ref[...], 0.0)
        logits = jnp.dot(hr, wcls_ref[...], preferred_element_type=jnp.float32) + bcls_ref[...]
        logits_ref[...] = logits
        lm = jnp.max(logits, axis=1, keepdims=True)
        le = jnp.exp(logits - lm)
        prob_ref[...] = le / jnp.sum(le, axis=1, keepdims=True)
        lane4 = lax.broadcasted_iota(jnp.int32, (1, 4), 1)
        yhat_ref[...] = jnp.min(
            jnp.where(logits >= lm, lane4, 9999), axis=1, keepdims=True)


def _run_tc(x_sorted, blk_cid, blk_valid, counts16, phi_W1, phi_b1, phi_W2,
            phi_b2, W_fc, b_fc, W_a, b_a, W_b, b_b, W_c, b_c, W_rho, b_rho,
            W_cls, b_cls):
    full = lambda *shape: pl.BlockSpec(shape, lambda i, *_: (0,) * len(shape))
    grid_spec = pltpu.PrefetchScalarGridSpec(
        num_scalar_prefetch=2,
        grid=(NBLK,),
        in_specs=[
            pl.BlockSpec((BLK, DIN), lambda i, *_: (i, 0)),
            full(NCLUST, DIN, DHID),
            full(NCLUST, DHID),
            full(NCLUST, DHID, DHID),
            full(NCLUST, DHID),
            full(16, 1),
            full(DHID, DHID), full(1, DHID),
            full(DHID, DATT), full(1, DATT),
            full(DHID, DATT), full(1, DATT),
            full(1, DATT), full(1, 1),
            full(DHID, DATT), full(1, DATT),
            full(DATT, 4), full(1, 4),
        ],
        out_specs=[full(1, 4), full(1, 4), full(1, 1)],
        scratch_shapes=[pltpu.VMEM((16, DHID), jnp.float32)],
    )
    return pl.pallas_call(
        _tc_kernel,
        grid_spec=grid_spec,
        out_shape=[
            jax.ShapeDtypeStruct((1, 4), jnp.float32),
            jax.ShapeDtypeStruct((1, 4), jnp.float32),
            jax.ShapeDtypeStruct((1, 1), jnp.int32),
        ],
    )(blk_cid, blk_valid, x_sorted, phi_W1, phi_b1, phi_W2, phi_b2, counts16,
      W_fc, b_fc.reshape(1, DHID), W_a, b_a.reshape(1, DATT),
      W_b, b_b.reshape(1, DATT), W_c.reshape(1, DATT), b_c.reshape(1, 1),
      W_rho, b_rho.reshape(1, DATT), W_cls, b_cls.reshape(1, 4))


ROWS_W = NPAD // 32
GCH = 104
NGCH = ROWS_W // GCH


def _sc_gather_body(data_hbm, perm_hbm, out_hbm, idx_v, rows_v, sem):
    wid = lax.axis_index("s") * 2 + lax.axis_index("c")
    base = wid * ROWS_W
    pltpu.sync_copy(perm_hbm.at[pl.ds(base, ROWS_W)], idx_v)

    def clamp(k, _):
        o = pl.multiple_of(k * 16, 16)
        v = idx_v[pl.ds(o, 16)]
        idx_v[pl.ds(o, 16)] = jnp.clip(v, 0, NTOK - 1)
        return 0
    lax.fori_loop(0, ROWS_W // 16, clamp, 0)

    def gather(j, _):
        o = pl.multiple_of(j * GCH, 8)
        pltpu.async_copy(data_hbm.at[idx_v.at[pl.ds(o, GCH)]], rows_v, sem).wait()
        pltpu.sync_copy(rows_v, out_hbm.at[pl.ds(base + o, GCH)])
        return 0
    lax.fori_loop(0, NGCH, gather, 0)


def _run_sc_gather(data, perm):
    mesh = plsc.VectorSubcoreMesh(core_axis_name="c", subcore_axis_name="s")
    return pl.kernel(
        _sc_gather_body,
        out_type=jax.ShapeDtypeStruct((NPAD, DIN), jnp.float32),
        mesh=mesh,
        scratch_types=[
            pltpu.VMEM((ROWS_W,), jnp.int32),
            pltpu.VMEM((GCH, DIN), jnp.float32),
            pltpu.SemaphoreType.DMA,
        ],
    )(data, perm)


def _route_host(cluster_id):
    cid = cluster_id.astype(jnp.int32)
    counts = jnp.zeros((NCLUST,), jnp.int32).at[cid].add(1)
    padded = ((counts + BLK - 1) // BLK) * BLK
    offsets = jnp.concatenate([jnp.zeros((1,), jnp.int32), jnp.cumsum(padded)])
    order = jnp.argsort(cid, stable=True)
    sorted_start = jnp.concatenate([jnp.zeros((1,), jnp.int32),
                                    jnp.cumsum(counts)])
    cid_sorted = cid[order]
    dest = (offsets[cid_sorted] + jnp.arange(NTOK, dtype=jnp.int32)
            - sorted_start[cid_sorted])
    perm = jnp.zeros((NPAD,), jnp.int32).at[dest].set(order.astype(jnp.int32))
    jb = jnp.arange(NBLK, dtype=jnp.int32) * BLK
    blk_cid = jnp.sum((jb[:, None] >= offsets[None, 1:]).astype(jnp.int32),
                      axis=1)
    blk_cid = jnp.minimum(blk_cid, NCLUST - 1)
    blk_valid = jnp.clip(counts[blk_cid] - (jb - offsets[blk_cid]), 0, BLK)
    counts16 = jnp.where(jnp.arange(16) < NCLUST,
                         jnp.concatenate([counts, jnp.ones((6,), jnp.int32)]),
                         1).astype(jnp.float32).reshape(16, 1)
    return perm, blk_cid, blk_valid, counts16


def kernel(data, cluster_id, phi_W1, phi_b1, phi_W2, phi_b2, W_fc, b_fc,
           W_a, b_a, W_b, b_b, W_c, b_c, W_rho, b_rho, W_cls, b_cls):
    perm, blk_cid, blk_valid, counts16 = _route_host(cluster_id)
    x_sorted = _run_sc_gather(data, perm)
    logits, prob, yhat = _run_tc(
        x_sorted, blk_cid, blk_valid, counts16, phi_W1, phi_b1, phi_W2,
        phi_b2, W_fc, b_fc, W_a, b_a, W_b, b_b, W_c, b_c, W_rho, b_rho,
        W_cls, b_cls)
    return (logits, prob, yhat)
